# baseline (speedup 1.0000x reference)
import functools

import jax
import jax.numpy as jnp
from jax import lax
from jax.experimental import pallas as pl
from jax.experimental.pallas import tpu as pltpu
from jax.experimental.pallas import tpu_sc as plsc

N = 10000
E = 320000
F = 128
U = 128

NC = 2
NS = 16
LANES = 16

ROWS_A = 624
ROWS_B = N - 15 * ROWS_A
NPAD = 10240
DEG_PER_TILE = NPAD // NS

EW = 128
TNW = E // EW
WPT = TNW // (NC * NS)
NTAIL = TNW - WPT * NC * NS

DR = 8
DK = 4
DNGRP = WPT // DR
DREM = WPT - DNGRP * DR

PR = 4
PK = 3
PNGRP = WPT // PR
PREM = WPT - PNGRP * PR

_mesh = plsc.VectorSubcoreMesh(core_axis_name="c", subcore_axis_name="s")


@functools.partial(
    pl.kernel,
    mesh=_mesh,
    out_type=jax.ShapeDtypeStruct((NC * NPAD,), jnp.float32),
    scratch_types=[
        pltpu.VMEM((DR, 2, EW), jnp.int32),
        pltpu.VMEM((EW,), jnp.float32),
        pltpu.VMEM((DEG_PER_TILE,), jnp.float32),
        pltpu.VMEM_SHARED((NPAD,), jnp.float32),
        pltpu.SemaphoreType.DMA,
        pltpu.SemaphoreType.DMA,
    ],
)
def _deg_kernel(ei_hbm, out_hbm, e_v, ones_v, z_v, hist_sh, sem_e, sem_s):
    cid = lax.axis_index("c")
    sid = lax.axis_index("s")
    tid = cid * NS + sid
    wbase = tid * WPT

    one = jnp.ones((LANES,), jnp.float32)
    zero = jnp.zeros((LANES,), jnp.float32)
    for j in range(EW // LANES):
        ones_v[pl.ds(j * LANES, LANES)] = one

    def _zwrite(i, carry):
        z_v[pl.ds(i * LANES, LANES)] = zero
        return carry

    lax.fori_loop(0, DEG_PER_TILE // LANES, _zwrite, 0)
    pltpu.sync_copy(z_v, hist_sh.at[pl.ds(sid * DEG_PER_TILE, DEG_PER_TILE)])
    plsc.subcore_barrier()

    for b in range(DK):
        pltpu.async_copy(ei_hbm.at[:, pl.ds((wbase + b) * EW, EW)],
                         e_v.at[b], sem_e)

    def _step(w, b):
        pltpu.make_async_copy(ei_hbm.at[:, pl.ds(0, EW)], e_v.at[b],
                              sem_e).wait()

        @pl.when(w >= DK)
        def _():
            pltpu.make_async_copy(ones_v, hist_sh.at[e_v.at[0, 0]],
                                  sem_s).wait()

        @pl.when(w + DK < WPT)
        def _():
            pltpu.async_copy(
                ei_hbm.at[:, pl.ds((wbase + w + DK) * EW, EW)],
                e_v.at[(b + DK) % DR], sem_e)

        pltpu.async_copy(ones_v, hist_sh.at[e_v.at[b, 0]], sem_s, add=True)

    def _grp(o, carry):
        for b in range(DR):
            _step(o * DR + b, b)
        return carry

    lax.fori_loop(0, DNGRP, _grp, 0)
    for t in range(DREM):
        _step(DNGRP * DR + t, t)

    def _drain(w, carry):
        pltpu.make_async_copy(ones_v, hist_sh.at[e_v.at[0, 0]], sem_s).wait()
        return carry

    lax.fori_loop(0, DK, _drain, 0)

    @pl.when(sid < NTAIL)
    def _():
        @pl.when(cid == 0)
        def _():
            pltpu.sync_copy(
                ei_hbm.at[:, pl.ds((NC * NS * WPT + sid) * EW, EW)],
                e_v.at[0])
            pltpu.sync_copy(ones_v, hist_sh.at[e_v.at[0, 0]], add=True)

    plsc.subcore_barrier()

    pltpu.sync_copy(
        hist_sh.at[pl.ds(sid * DEG_PER_TILE, DEG_PER_TILE)],
        out_hbm.at[pl.ds(cid * NPAD + sid * DEG_PER_TILE, DEG_PER_TILE)],
    )


@functools.partial(
    pl.kernel,
    mesh=_mesh,
    out_type=jax.ShapeDtypeStruct((NC, N, U), jnp.float32),
    scratch_types=[
        pltpu.VMEM((PR, 2, EW), jnp.int32),
        pltpu.VMEM((2, EW, U), jnp.float32),
        pltpu.VMEM_SHARED((N, U), jnp.float32),
        pltpu.SemaphoreType.DMA,
        pltpu.SemaphoreType.DMA,
    ],
)
def _prop_kernel(h2_hbm, ei_hbm, out_hbm, e_v, g_v, acc_sh, sem_g, sem_e):
    cid = lax.axis_index("c")
    sid = lax.axis_index("s")
    tid = cid * NS + sid
    wbase = tid * WPT
    rbase = sid * ROWS_A

    @pl.when(cid == 0)
    def _():
        @pl.when(sid < NS - 1)
        def _():
            pltpu.sync_copy(h2_hbm.at[pl.ds(rbase, ROWS_A)],
                            acc_sh.at[pl.ds(rbase, ROWS_A)])

        @pl.when(sid == NS - 1)
        def _():
            pltpu.sync_copy(h2_hbm.at[pl.ds(15 * ROWS_A, ROWS_B)],
                            acc_sh.at[pl.ds(15 * ROWS_A, ROWS_B)])

    @pl.when(cid == 1)
    def _():
        def _zrow(r, carry):
            for j in range(U // LANES):
                g_v[0, r, pl.ds(j * LANES, LANES)] = jnp.zeros(
                    (LANES,), jnp.float32)
            return carry

        lax.fori_loop(0, EW, _zrow, 0)
        nfull = ROWS_A // EW
        for k in range(nfull):
            pltpu.sync_copy(g_v.at[0],
                            acc_sh.at[pl.ds(rbase + k * EW, EW)])

        @pl.when(sid < NS - 1)
        def _():
            pltpu.sync_copy(g_v.at[0, pl.ds(0, ROWS_A - nfull * EW)],
                            acc_sh.at[pl.ds(rbase + nfull * EW,
                                            ROWS_A - nfull * EW)])

        @pl.when(sid == NS - 1)
        def _():
            pltpu.sync_copy(g_v.at[0],
                            acc_sh.at[pl.ds(rbase + nfull * EW, EW)])

    plsc.subcore_barrier()

    for b in range(PK):
        pltpu.async_copy(ei_hbm.at[:, pl.ds((wbase + b) * EW, EW)],
                         e_v.at[b], sem_e)
    pltpu.make_async_copy(ei_hbm.at[:, pl.ds(0, EW)], e_v.at[0],
                          sem_e).wait()
    pltpu.async_copy(h2_hbm.at[e_v.at[0, 0]], g_v.at[0], sem_g)

    def _step(w, b):
        pltpu.make_async_copy(h2_hbm.at[e_v.at[0, 0]], g_v.at[w % 2],
                              sem_g).wait()

        @pl.when(w + 1 < WPT)
        def _():
            pltpu.make_async_copy(ei_hbm.at[:, pl.ds(0, EW)],
                                  e_v.at[(b + 1) % PR], sem_e).wait()
            pltpu.async_copy(h2_hbm.at[e_v.at[(b + 1) % PR, 0]],
                             g_v.at[(w + 1) % 2], sem_g)

        pltpu.sync_copy(g_v.at[w % 2], acc_sh.at[e_v.at[b, 1]], add=True)

        @pl.when(w + PK < WPT)
        def _():
            pltpu.async_copy(
                ei_hbm.at[:, pl.ds((wbase + w + PK) * EW, EW)],
                e_v.at[(b + PK) % PR], sem_e)

    def _grp(o, carry):
        for b in range(PR):
            _step(o * PR + b, b)
        return carry

    lax.fori_loop(0, PNGRP, _grp, 0)
    for t in range(PREM):
        _step(PNGRP * PR + t, t)

    @pl.when(sid < NTAIL)
    def _():
        @pl.when(cid == 1)
        def _():
            pltpu.sync_copy(
                ei_hbm.at[:, pl.ds((NC * NS * WPT + sid) * EW, EW)],
                e_v.at[0])
            pltpu.async_copy(h2_hbm.at[e_v.at[0, 0]], g_v.at[0],
                             sem_g).wait()
            pltpu.sync_copy(g_v.at[0], acc_sh.at[e_v.at[0, 1]], add=True)

    plsc.subcore_barrier()

    @pl.when(sid < NS - 1)
    def _():
        pltpu.sync_copy(acc_sh.at[pl.ds(rbase, ROWS_A)],
                        out_hbm.at[cid, pl.ds(rbase, ROWS_A)])

    @pl.when(sid == NS - 1)
    def _():
        pltpu.sync_copy(acc_sh.at[pl.ds(15 * ROWS_A, ROWS_B)],
                        out_hbm.at[cid, pl.ds(15 * ROWS_A, ROWS_B)])


def _h2_body(x_ref, w_ref, d_ref, h2_ref):
    dinv = lax.rsqrt(d_ref[...])
    h = jnp.dot(x_ref[...], w_ref[...], preferred_element_type=jnp.float32)
    h2_ref[...] = h * dinv


def _combine_body(p0_ref, p1_ref, d_ref, b_ref, o_ref):
    dinv = lax.rsqrt(d_ref[...])
    o_ref[...] = (p0_ref[0] + p1_ref[0]) * dinv + b_ref[...]


_BLK = 2000


def kernel(x, edge_index, kernel, bias):
    deg_part = _deg_kernel(edge_index)
    d = (deg_part[:N] + deg_part[NPAD:NPAD + N] + 1.0).reshape(N, 1)

    grid = N // _BLK
    h2 = pl.pallas_call(
        _h2_body,
        grid=(grid,),
        in_specs=[
            pl.BlockSpec((_BLK, F), lambda i: (i, 0)),
            pl.BlockSpec((F, U), lambda i: (0, 0)),
            pl.BlockSpec((_BLK, 1), lambda i: (i, 0)),
        ],
        out_specs=pl.BlockSpec((_BLK, U), lambda i: (i, 0)),
        out_shape=jax.ShapeDtypeStruct((N, U), jnp.float32),
    )(x, kernel, d)

    p = _prop_kernel(h2, edge_index)

    out = pl.pallas_call(
        _combine_body,
        grid=(grid,),
        in_specs=[
            pl.BlockSpec((1, _BLK, U), lambda i: (0, i, 0)),
            pl.BlockSpec((1, _BLK, U), lambda i: (1, i, 0)),
            pl.BlockSpec((_BLK, 1), lambda i: (i, 0)),
            pl.BlockSpec((1, U), lambda i: (0, 0)),
        ],
        out_specs=pl.BlockSpec((_BLK, U), lambda i: (i, 0)),
        out_shape=jax.ShapeDtypeStruct((N, U), jnp.float32),
    )(p, p, d, bias.reshape(1, U))
    return out


# speedup vs baseline: 1.1882x; 1.1882x over previous
import functools

import jax
import jax.numpy as jnp
from jax import lax
from jax.experimental import pallas as pl
from jax.experimental.pallas import tpu as pltpu
from jax.experimental.pallas import tpu_sc as plsc

N = 10000
E = 320000
F = 128
U = 128

NC = 2
NS = 16
LANES = 16

E_PER_TILE = E // (NC * NS)
WIN = 80
NWIN = E_PER_TILE // WIN
ROWS_A = 624
ROWS_B = N - 15 * ROWS_A
NPAD = 10240
DEG_PER_TILE = NPAD // NS

GRP = 5
PWIN = 40
PNWIN = E_PER_TILE // PWIN
NGRP = PNWIN // GRP

_mesh = plsc.VectorSubcoreMesh(core_axis_name="c", subcore_axis_name="s")


@functools.partial(
    pl.kernel,
    mesh=_mesh,
    out_type=jax.ShapeDtypeStruct((NC * NPAD,), jnp.float32),
    scratch_types=[
        pltpu.VMEM((E_PER_TILE,), jnp.int32),
        pltpu.VMEM((WIN,), jnp.float32),
        pltpu.VMEM((DEG_PER_TILE,), jnp.float32),
        pltpu.VMEM_SHARED((NPAD,), jnp.float32),
        pltpu.SemaphoreType.DMA,
    ],
)
def _deg_kernel(row_hbm, out_hbm, idx_v, ones_v, z_v, hist_sh, sem):
    cid = lax.axis_index("c")
    sid = lax.axis_index("s")
    tid = cid * NS + sid

    one = jnp.ones((LANES,), jnp.float32)
    zero = jnp.zeros((LANES,), jnp.float32)
    for j in range(WIN // LANES):
        ones_v[pl.ds(j * LANES, LANES)] = one

    def _zwrite(i, carry):
        z_v[pl.ds(i * LANES, LANES)] = zero
        return carry

    lax.fori_loop(0, DEG_PER_TILE // LANES, _zwrite, 0)
    pltpu.sync_copy(z_v, hist_sh.at[pl.ds(sid * DEG_PER_TILE, DEG_PER_TILE)])
    pltpu.sync_copy(row_hbm.at[pl.ds(tid * E_PER_TILE, E_PER_TILE)], idx_v)
    plsc.subcore_barrier()

    def _win(w, carry):
        pltpu.async_copy(ones_v, hist_sh.at[idx_v.at[pl.ds(w * WIN, WIN)]],
                         sem, add=True)
        return carry

    lax.fori_loop(0, NWIN, _win, 0)

    def _drain(w, carry):
        pltpu.make_async_copy(ones_v, hist_sh.at[idx_v.at[pl.ds(0, WIN)]],
                              sem).wait()
        return carry

    lax.fori_loop(0, NWIN, _drain, 0)
    plsc.subcore_barrier()

    pltpu.sync_copy(
        hist_sh.at[pl.ds(sid * DEG_PER_TILE, DEG_PER_TILE)],
        out_hbm.at[pl.ds(cid * NPAD + sid * DEG_PER_TILE, DEG_PER_TILE)],
    )


@functools.partial(
    pl.kernel,
    mesh=_mesh,
    out_type=jax.ShapeDtypeStruct((NC, N, U), jnp.float32),
    scratch_types=[
        pltpu.VMEM((E_PER_TILE,), jnp.int32),
        pltpu.VMEM((GRP, PWIN), jnp.int32),
        pltpu.VMEM((GRP, PWIN, U), jnp.float32),
        pltpu.VMEM_SHARED((N, U), jnp.float32),
        pltpu.SemaphoreType.DMA,
        pltpu.SemaphoreType.DMA,
    ],
)
def _prop_kernel(h2_hbm, row_hbm, col_hbm, out_hbm, idxr_v, idxc_v, g_v,
                 acc_sh, sem, sem_c):
    cid = lax.axis_index("c")
    sid = lax.axis_index("s")
    tid = cid * NS + sid
    rbase = sid * ROWS_A

    @pl.when(cid == 0)
    def _():
        @pl.when(sid < NS - 1)
        def _():
            pltpu.sync_copy(h2_hbm.at[pl.ds(rbase, ROWS_A)],
                            acc_sh.at[pl.ds(rbase, ROWS_A)])

        @pl.when(sid == NS - 1)
        def _():
            pltpu.sync_copy(h2_hbm.at[pl.ds(15 * ROWS_A, ROWS_B)],
                            acc_sh.at[pl.ds(15 * ROWS_A, ROWS_B)])

    @pl.when(cid == 1)
    def _():
        def _zrow(r, carry):
            for j in range(U // LANES):
                g_v[0, r, pl.ds(j * LANES, LANES)] = jnp.zeros(
                    (LANES,), jnp.float32)
            return carry

        lax.fori_loop(0, PWIN, _zrow, 0)
        nfull = ROWS_A // PWIN
        for k in range(nfull):
            pltpu.sync_copy(g_v.at[0],
                            acc_sh.at[pl.ds(rbase + k * PWIN, PWIN)])

        @pl.when(sid < NS - 1)
        def _():
            pltpu.sync_copy(g_v.at[0, pl.ds(0, ROWS_A - nfull * PWIN)],
                            acc_sh.at[pl.ds(rbase + nfull * PWIN,
                                            ROWS_A - nfull * PWIN)])

        @pl.when(sid == NS - 1)
        def _():
            pltpu.sync_copy(g_v.at[0],
                            acc_sh.at[pl.ds(rbase + nfull * PWIN, PWIN)])

    pltpu.sync_copy(row_hbm.at[pl.ds(tid * E_PER_TILE, E_PER_TILE)], idxr_v)
    plsc.subcore_barrier()

    cbase = tid * E_PER_TILE
    for b in range(GRP):
        pltpu.async_copy(col_hbm.at[pl.ds(cbase + b * PWIN, PWIN)],
                         idxc_v.at[b], sem_c)
        pltpu.async_copy(h2_hbm.at[idxr_v.at[pl.ds(b * PWIN, PWIN)]],
                         g_v.at[b], sem)

    def _step(w, b):
        pltpu.make_async_copy(col_hbm.at[pl.ds(cbase, PWIN)], idxc_v.at[b],
                              sem_c).wait()
        pltpu.make_async_copy(h2_hbm.at[idxr_v.at[pl.ds(0, PWIN)]],
                              g_v.at[b], sem).wait()
        pltpu.sync_copy(g_v.at[b], acc_sh.at[idxc_v.at[b]], add=True)

        @pl.when(w + GRP < PNWIN)
        def _():
            pltpu.async_copy(
                col_hbm.at[pl.ds(cbase + (w + GRP) * PWIN, PWIN)],
                idxc_v.at[b], sem_c)
            pltpu.async_copy(
                h2_hbm.at[idxr_v.at[pl.ds((w + GRP) * PWIN, PWIN)]],
                g_v.at[b], sem)

    def _grp(o, carry):
        for b in range(GRP):
            _step(o * GRP + b, b)
        return carry

    lax.fori_loop(0, NGRP, _grp, 0)
    plsc.subcore_barrier()

    @pl.when(sid < NS - 1)
    def _():
        pltpu.sync_copy(acc_sh.at[pl.ds(rbase, ROWS_A)],
                        out_hbm.at[cid, pl.ds(rbase, ROWS_A)])

    @pl.when(sid == NS - 1)
    def _():
        pltpu.sync_copy(acc_sh.at[pl.ds(15 * ROWS_A, ROWS_B)],
                        out_hbm.at[cid, pl.ds(15 * ROWS_A, ROWS_B)])


def _h2_body(x_ref, w_ref, d_ref, h2_ref):
    dinv = lax.rsqrt(d_ref[...])
    h = jnp.dot(x_ref[...], w_ref[...], preferred_element_type=jnp.float32)
    h2_ref[...] = h * dinv


def _combine_body(p0_ref, p1_ref, d_ref, b_ref, o_ref):
    dinv = lax.rsqrt(d_ref[...])
    o_ref[...] = (p0_ref[0] + p1_ref[0]) * dinv + b_ref[...]


_BLK = 2000


def kernel(x, edge_index, kernel, bias):
    row = edge_index[0]
    col = lax.optimization_barrier(edge_index)[1]

    deg_part = _deg_kernel(row)
    d = (deg_part[:N] + deg_part[NPAD:NPAD + N] + 1.0).reshape(N, 1)

    grid = N // _BLK
    h2 = pl.pallas_call(
        _h2_body,
        grid=(grid,),
        in_specs=[
            pl.BlockSpec((_BLK, F), lambda i: (i, 0)),
            pl.BlockSpec((F, U), lambda i: (0, 0)),
            pl.BlockSpec((_BLK, 1), lambda i: (i, 0)),
        ],
        out_specs=pl.BlockSpec((_BLK, U), lambda i: (i, 0)),
        out_shape=jax.ShapeDtypeStruct((N, U), jnp.float32),
    )(x, kernel, d)

    p = _prop_kernel(h2, row, col)

    out = pl.pallas_call(
        _combine_body,
        grid=(grid,),
        in_specs=[
            pl.BlockSpec((1, _BLK, U), lambda i: (0, i, 0)),
            pl.BlockSpec((1, _BLK, U), lambda i: (1, i, 0)),
            pl.BlockSpec((_BLK, 1), lambda i: (i, 0)),
            pl.BlockSpec((1, U), lambda i: (0, 0)),
        ],
        out_specs=pl.BlockSpec((_BLK, U), lambda i: (i, 0)),
        out_shape=jax.ShapeDtypeStruct((N, U), jnp.float32),
    )(p, p, d, bias.reshape(1, U))
    return out
